# Initial kernel scaffold; baseline (speedup 1.0000x reference)
import jax
import jax.numpy as jnp
from jax.experimental import pallas as pl


def kernel(x, edge_index, W, b):
    raise NotImplementedError("write your pallas kernel here")


# speedup vs baseline: 23.8467x; 23.8467x over previous
import functools

import jax
import jax.numpy as jnp
from jax import lax
from jax.experimental import pallas as pl
from jax.experimental.pallas import tpu as pltpu
from jax.experimental.pallas import tpu_sc as plsc

C_LANES = 128
CH = C_LANES // 2
NC = 2
NS = 16
B = 128
DW = 16
RB = 256


def _matmul_body(x_ref, w_ref, h_ref):
    h_ref[...] = jnp.dot(x_ref[...], w_ref[...],
                         preferred_element_type=jnp.float32)


def _tc_matmul(x_pad, w):
    np_rows = x_pad.shape[0]
    return pl.pallas_call(
        _matmul_body,
        grid=(np_rows // RB,),
        in_specs=[pl.BlockSpec((RB, C_LANES), lambda i: (i, 0)),
                  pl.BlockSpec((C_LANES, C_LANES), lambda i: (0, 0))],
        out_specs=pl.BlockSpec((RB, C_LANES), lambda i: (i, 0)),
        out_shape=jax.ShapeDtypeStruct((np_rows, C_LANES), jnp.float32),
    )(x_pad, w)


def _sc_degree(dst4, np_rows):
    kb = dst4.shape[2]
    stripe = np_rows // NS
    mesh = plsc.VectorSubcoreMesh(core_axis_name="c", subcore_axis_name="s")

    @functools.partial(
        pl.kernel,
        out_type=jax.ShapeDtypeStruct((NC, np_rows, DW), jnp.float32),
        mesh=mesh,
        scratch_types=[
            pltpu.VMEM((kb, B), jnp.int32),
            pltpu.VMEM((B, DW), jnp.float32),
            pltpu.VMEM((stripe, DW), jnp.float32),
            pltpu.VMEM_SHARED((np_rows, DW), jnp.float32),
        ],
        compiler_params=pltpu.CompilerParams(use_tc_tiling_on_sc=False),
    )
    def k(dst_hbm, out_hbm, idx_v, ones_v, z_v, acc_sh):
        cid = lax.axis_index("c")
        sid = lax.axis_index("s")

        def init_ones(i, carry):
            ones_v[i, :] = jnp.full((DW,), 1.0, jnp.float32)
            return carry
        lax.fori_loop(0, B, init_ones, 0)

        def init_zeros(i, carry):
            z_v[i, :] = jnp.zeros((DW,), jnp.float32)
            return carry
        lax.fori_loop(0, stripe, init_zeros, 0)

        pltpu.sync_copy(z_v, acc_sh.at[pl.ds(sid * stripe, stripe)])
        plsc.subcore_barrier()

        pltpu.sync_copy(dst_hbm.at[cid, sid], idx_v)

        def body(bi, carry):
            pltpu.sync_copy(ones_v, acc_sh.at[idx_v.at[bi]], add=True)
            return carry
        lax.fori_loop(0, kb, body, 0)

        plsc.subcore_barrier()
        pltpu.sync_copy(acc_sh.at[pl.ds(sid * stripe, stripe)],
                        out_hbm.at[cid, pl.ds(sid * stripe, stripe)])

    return k(dst4)


def _scale_body(h_ref, d0_ref, d1_ref, g2_ref, dis_ref):
    deg = d0_ref[:, :1] + d1_ref[:, :1] + 1.0
    dis = lax.rsqrt(deg)
    half = jnp.where(pl.program_id(1) == 0,
                     h_ref[:, :CH], h_ref[:, CH:])
    g2_ref[0] = half * dis
    dis_ref[...] = jnp.broadcast_to(dis, dis_ref.shape)


def _tc_scale(h, deg0, deg1):
    np_rows = h.shape[0]
    return pl.pallas_call(
        _scale_body,
        grid=(np_rows // RB, NC),
        in_specs=[pl.BlockSpec((RB, C_LANES), lambda i, c: (i, 0)),
                  pl.BlockSpec((RB, DW), lambda i, c: (i, 0)),
                  pl.BlockSpec((RB, DW), lambda i, c: (i, 0))],
        out_specs=[pl.BlockSpec((1, RB, CH), lambda i, c: (c, i, 0)),
                   pl.BlockSpec((RB, 8), lambda i, c: (i, 0))],
        out_shape=[jax.ShapeDtypeStruct((NC, np_rows, CH), jnp.float32),
                   jax.ShapeDtypeStruct((np_rows, 8), jnp.float32)],
    )(h, deg0, deg1)


def _sc_messages(g2flat, src3, dst3, np_rows):
    kb = src3.shape[1]
    stripe = np_rows // NS
    n_init = stripe // B
    mesh = plsc.VectorSubcoreMesh(core_axis_name="c", subcore_axis_name="s")

    @functools.partial(
        pl.kernel,
        out_type=jax.ShapeDtypeStruct((NC, np_rows, CH), jnp.float32),
        mesh=mesh,
        scratch_types=[
            pltpu.VMEM((kb, B), jnp.int32),
            pltpu.VMEM((kb, B), jnp.int32),
            pltpu.VMEM((B, CH), jnp.float32),
            pltpu.VMEM((B, CH), jnp.float32),
            pltpu.VMEM_SHARED((np_rows, CH), jnp.float32),
            pltpu.SemaphoreType.DMA,
            pltpu.SemaphoreType.DMA,
        ],
        compiler_params=pltpu.CompilerParams(use_tc_tiling_on_sc=False),
    )
    def k(g_hbm, src_hbm, dst_hbm, out_hbm,
          src_v, dst_v, r0, r1, acc_sh, sem0, sem1):
        cid = lax.axis_index("c")
        sid = lax.axis_index("s")

        def zrow(i, carry):
            for j in range(CH // 16):
                r0[i, pl.ds(j * 16, 16)] = jnp.zeros((16,), jnp.float32)
            return carry
        lax.fori_loop(0, B, zrow, 0)
        for t in range(n_init):
            pltpu.sync_copy(r0, acc_sh.at[pl.ds(sid * stripe + t * B, B)])
        plsc.subcore_barrier()

        pltpu.sync_copy(src_hbm.at[sid], src_v)
        pltpu.sync_copy(dst_hbm.at[sid], dst_v)

        off = (cid * np_rows).astype(jnp.int32)
        def adj(i, carry):
            for j in range(B // 16):
                sl = pl.ds(j * 16, 16)
                src_v[i, sl] = src_v[i, sl] + off
            return carry
        lax.fori_loop(0, kb, adj, 0)

        pltpu.async_copy(g_hbm.at[src_v.at[0]], r0, sem0)

        def body(gi, carry):
            b0 = 2 * gi
            b1 = b0 + 1
            pltpu.async_copy(g_hbm.at[src_v.at[b1]], r1, sem1)
            pltpu.make_async_copy(g_hbm.at[src_v.at[b0]], r0, sem0).wait()
            pltpu.sync_copy(r0, acc_sh.at[dst_v.at[b0]], add=True)

            @pl.when(gi < kb // 2 - 1)
            def _():
                pltpu.async_copy(g_hbm.at[src_v.at[b0 + 2]], r0, sem0)

            pltpu.make_async_copy(g_hbm.at[src_v.at[b1]], r1, sem1).wait()
            pltpu.sync_copy(r1, acc_sh.at[dst_v.at[b1]], add=True)
            return carry
        lax.fori_loop(0, kb // 2, body, 0)

        plsc.subcore_barrier()
        pltpu.sync_copy(acc_sh.at[pl.ds(sid * stripe, stripe)],
                        out_hbm.at[cid, pl.ds(sid * stripe, stripe)])

    return k(g2flat, src3, dst3)


def _final_body(q_ref, g_ref, dis_ref, b_ref, o_ref):
    s = dis_ref[:, :1]
    acc = (q_ref[0] + g_ref[0]) * s + b_ref[0, :1]
    o_ref[0] = jnp.maximum(acc, 0.0)


def _tc_final(q2, g2, dis, bias2):
    np_rows = g2.shape[1]
    return pl.pallas_call(
        _final_body,
        grid=(np_rows // RB, NC),
        in_specs=[pl.BlockSpec((1, RB, CH), lambda i, c: (c, i, 0)),
                  pl.BlockSpec((1, RB, CH), lambda i, c: (c, i, 0)),
                  pl.BlockSpec((RB, 8), lambda i, c: (i, 0)),
                  pl.BlockSpec((1, 8, CH), lambda i, c: (c, 0, 0))],
        out_specs=pl.BlockSpec((1, RB, CH), lambda i, c: (c, i, 0)),
        out_shape=jax.ShapeDtypeStruct((NC, np_rows, CH), jnp.float32),
    )(q2, g2, dis, bias2)


def kernel(x, edge_index, W, b):
    n, c = x.shape
    e = edge_index.shape[1]
    assert c == C_LANES

    align = 2048
    np_rows = -(-(n + 1) // align) * align
    chunk = NS * B * 2
    e_pad = -(-e // chunk) * chunk

    src = edge_index[0].astype(jnp.int32)
    dst = edge_index[1].astype(jnp.int32)
    pad_len = e_pad - e
    if pad_len:
        pad_idx = n + (jnp.arange(pad_len, dtype=jnp.int32) % (np_rows - n))
        src = jnp.concatenate([src, pad_idx])
        dst = jnp.concatenate([dst, pad_idx])
    dst4 = dst.reshape(NC, NS, -1, B)
    src3 = src.reshape(NS, -1, B)
    dst3 = dst.reshape(NS, -1, B)

    x_pad = jnp.concatenate(
        [x, jnp.zeros((np_rows - n, c), jnp.float32)], axis=0)

    h = _tc_matmul(x_pad, W)
    deg = _sc_degree(dst4, np_rows)
    g2, dis = _tc_scale(h, deg[0], deg[1])
    q2 = _sc_messages(g2.reshape(NC * np_rows, CH), src3, dst3, np_rows)
    bias2 = jnp.broadcast_to(b.reshape(NC, 1, CH), (NC, 8, CH))
    out3 = _tc_final(q2, g2, dis, bias2)
    out = jnp.concatenate([out3[0], out3[1]], axis=1)
    return out[:n]


# speedup vs baseline: 30.1166x; 1.2629x over previous
import functools

import jax
import jax.numpy as jnp
from jax import lax
from jax.experimental import pallas as pl
from jax.experimental.pallas import tpu as pltpu
from jax.experimental.pallas import tpu_sc as plsc

C_LANES = 128
CH = C_LANES // 2
NC = 2
NS = 16
B = 128
DW = 16
RB = 256


def _gscale_body(x_ref, w_ref, d0_ref, d1_ref, g2_ref, dis_ref):
    h = jnp.dot(x_ref[...], w_ref[...], preferred_element_type=jnp.float32)
    deg = d0_ref[:, :1] + d1_ref[:, :1] + 1.0
    dis = lax.rsqrt(deg)
    g2_ref[0] = h[:, :CH] * dis
    g2_ref[1] = h[:, CH:] * dis
    dis_ref[...] = jnp.broadcast_to(dis, dis_ref.shape)


def _tc_gscale(x_pad, w, deg0, deg1):
    np_rows = x_pad.shape[0]
    return pl.pallas_call(
        _gscale_body,
        grid=(np_rows // RB,),
        in_specs=[pl.BlockSpec((RB, C_LANES), lambda i: (i, 0)),
                  pl.BlockSpec((C_LANES, C_LANES), lambda i: (0, 0)),
                  pl.BlockSpec((RB, DW), lambda i: (i, 0)),
                  pl.BlockSpec((RB, DW), lambda i: (i, 0))],
        out_specs=[pl.BlockSpec((NC, RB, CH), lambda i: (0, i, 0)),
                   pl.BlockSpec((RB, 8), lambda i: (i, 0))],
        out_shape=[jax.ShapeDtypeStruct((NC, np_rows, CH), jnp.float32),
                   jax.ShapeDtypeStruct((np_rows, 8), jnp.float32)],
    )(x_pad, w, deg0, deg1)


def _sc_degree(dst4, np_rows):
    kb = dst4.shape[2]
    stripe = np_rows // NS
    mesh = plsc.VectorSubcoreMesh(core_axis_name="c", subcore_axis_name="s")

    @functools.partial(
        pl.kernel,
        out_type=jax.ShapeDtypeStruct((NC, np_rows, DW), jnp.float32),
        mesh=mesh,
        scratch_types=[
            pltpu.VMEM((kb, B), jnp.int32),
            pltpu.VMEM((B, DW), jnp.float32),
            pltpu.VMEM((stripe, DW), jnp.float32),
            pltpu.VMEM_SHARED((np_rows, DW), jnp.float32),
        ],
        compiler_params=pltpu.CompilerParams(use_tc_tiling_on_sc=False),
    )
    def k(dst_hbm, out_hbm, idx_v, ones_v, z_v, acc_sh):
        cid = lax.axis_index("c")
        sid = lax.axis_index("s")

        def init_ones(i, carry):
            ones_v[i, :] = jnp.full((DW,), 1.0, jnp.float32)
            return carry
        lax.fori_loop(0, B, init_ones, 0)

        def init_zeros(i, carry):
            z_v[i, :] = jnp.zeros((DW,), jnp.float32)
            return carry
        lax.fori_loop(0, stripe, init_zeros, 0)

        pltpu.sync_copy(z_v, acc_sh.at[pl.ds(sid * stripe, stripe)])
        plsc.subcore_barrier()

        pltpu.sync_copy(dst_hbm.at[cid, sid], idx_v)

        def body(bi, carry):
            pltpu.sync_copy(ones_v, acc_sh.at[idx_v.at[bi]], add=True)
            return carry
        lax.fori_loop(0, kb, body, 0)

        plsc.subcore_barrier()
        pltpu.sync_copy(acc_sh.at[pl.ds(sid * stripe, stripe)],
                        out_hbm.at[cid, pl.ds(sid * stripe, stripe)])

    return k(dst4)


def _sc_messages(g2flat, src3, dst3, np_rows):
    kb = src3.shape[1]
    stripe = np_rows // NS
    n_init = stripe // B
    mesh = plsc.VectorSubcoreMesh(core_axis_name="c", subcore_axis_name="s")

    @functools.partial(
        pl.kernel,
        out_type=jax.ShapeDtypeStruct((np_rows, C_LANES), jnp.float32),
        mesh=mesh,
        scratch_types=[
            pltpu.VMEM((kb, B), jnp.int32),
            pltpu.VMEM((kb, B), jnp.int32),
            pltpu.VMEM((B, CH), jnp.float32),
            pltpu.VMEM((B, CH), jnp.float32),
            pltpu.VMEM_SHARED((np_rows, CH), jnp.float32),
            pltpu.SemaphoreType.DMA,
            pltpu.SemaphoreType.DMA,
        ],
        compiler_params=pltpu.CompilerParams(use_tc_tiling_on_sc=False),
    )
    def k(g_hbm, src_hbm, dst_hbm, out_hbm,
          src_v, dst_v, r0, r1, acc_sh, sem0, sem1):
        cid = lax.axis_index("c")
        sid = lax.axis_index("s")

        def zrow(i, carry):
            for j in range(CH // 16):
                r0[i, pl.ds(j * 16, 16)] = jnp.zeros((16,), jnp.float32)
            return carry
        lax.fori_loop(0, B, zrow, 0)
        for t in range(n_init):
            pltpu.sync_copy(r0, acc_sh.at[pl.ds(sid * stripe + t * B, B)])
        plsc.subcore_barrier()

        pltpu.sync_copy(src_hbm.at[sid], src_v)
        pltpu.sync_copy(dst_hbm.at[sid], dst_v)

        off = (cid * np_rows).astype(jnp.int32)
        def adj(i, carry):
            for j in range(B // 16):
                sl = pl.ds(j * 16, 16)
                src_v[i, sl] = src_v[i, sl] + off
            return carry
        lax.fori_loop(0, kb, adj, 0)

        pltpu.async_copy(g_hbm.at[src_v.at[0]], r0, sem0)

        def body(gi, carry):
            b0 = 2 * gi
            b1 = b0 + 1
            pltpu.async_copy(g_hbm.at[src_v.at[b1]], r1, sem1)
            pltpu.make_async_copy(g_hbm.at[src_v.at[b0]], r0, sem0).wait()
            pltpu.sync_copy(r0, acc_sh.at[dst_v.at[b0]], add=True)

            @pl.when(gi < kb // 2 - 1)
            def _():
                pltpu.async_copy(g_hbm.at[src_v.at[b0 + 2]], r0, sem0)

            pltpu.make_async_copy(g_hbm.at[src_v.at[b1]], r1, sem1).wait()
            pltpu.sync_copy(r1, acc_sh.at[dst_v.at[b1]], add=True)
            return carry
        lax.fori_loop(0, kb // 2, body, 0)

        plsc.subcore_barrier()
        pltpu.sync_copy(
            acc_sh.at[pl.ds(sid * stripe, stripe)],
            out_hbm.at[pl.ds(sid * stripe, stripe), pl.ds(cid * CH, CH)])

    return k(g2flat, src3, dst3)


def _final_body(q_ref, g_ref, dis_ref, b_ref, o_ref):
    s = dis_ref[:, :1]
    full = jnp.concatenate(
        [q_ref[:, :CH] + g_ref[0], q_ref[:, CH:] + g_ref[1]], axis=1)
    o_ref[...] = jnp.maximum(full * s + b_ref[:1], 0.0)


def _tc_final(q, g2, dis, bias):
    np_rows = g2.shape[1]
    return pl.pallas_call(
        _final_body,
        grid=(np_rows // RB,),
        in_specs=[pl.BlockSpec((RB, C_LANES), lambda i: (i, 0)),
                  pl.BlockSpec((NC, RB, CH), lambda i: (0, i, 0)),
                  pl.BlockSpec((RB, 8), lambda i: (i, 0)),
                  pl.BlockSpec((8, C_LANES), lambda i: (0, 0))],
        out_specs=pl.BlockSpec((RB, C_LANES), lambda i: (i, 0)),
        out_shape=jax.ShapeDtypeStruct((np_rows, C_LANES), jnp.float32),
    )(q, g2, dis, bias)


def kernel(x, edge_index, W, b):
    n, c = x.shape
    e = edge_index.shape[1]
    assert c == C_LANES

    align = 2048
    np_rows = -(-(n + 1) // align) * align
    chunk = NS * B * 2
    e_pad = -(-e // chunk) * chunk

    src = edge_index[0].astype(jnp.int32)
    dst = edge_index[1].astype(jnp.int32)
    pad_len = e_pad - e
    if pad_len:
        pad_idx = n + (jnp.arange(pad_len, dtype=jnp.int32) % (np_rows - n))
        src = jnp.concatenate([src, pad_idx])
        dst = jnp.concatenate([dst, pad_idx])
    dst4 = dst.reshape(NC, NS, -1, B)
    src3 = src.reshape(NS, -1, B)
    dst3 = dst.reshape(NS, -1, B)

    x_pad = jnp.concatenate(
        [x, jnp.zeros((np_rows - n, c), jnp.float32)], axis=0)

    deg = _sc_degree(dst4, np_rows)
    g2, dis = _tc_gscale(x_pad, W, deg[0], deg[1])
    q = _sc_messages(g2.reshape(NC * np_rows, CH), src3, dst3, np_rows)
    bias = jnp.broadcast_to(b.reshape(1, C_LANES), (8, C_LANES))
    out = _tc_final(q, g2, dis, bias)
    return out[:n]


# speedup vs baseline: 36.0572x; 1.1973x over previous
import functools

import jax
import jax.numpy as jnp
from jax import lax
from jax.experimental import pallas as pl
from jax.experimental.pallas import tpu as pltpu
from jax.experimental.pallas import tpu_sc as plsc

C_LANES = 128
CH = C_LANES // 2
NC = 2
NS = 16
B = 128
DW = 16
RB = 1024
NBUF = 4


def _gscale_body(x_ref, w_ref, d0_ref, d1_ref, g2_ref, dis_ref):
    h = jnp.dot(x_ref[...], w_ref[...], preferred_element_type=jnp.float32)
    deg = d0_ref[:, :1] + d1_ref[:, :1] + 1.0
    dis = lax.rsqrt(deg)
    g2_ref[0] = h[:, :CH] * dis
    g2_ref[1] = h[:, CH:] * dis
    dis_ref[...] = jnp.broadcast_to(dis, dis_ref.shape)


def _tc_gscale(x_pad, w, deg0, deg1):
    np_rows = x_pad.shape[0]
    return pl.pallas_call(
        _gscale_body,
        grid=(np_rows // RB,),
        in_specs=[pl.BlockSpec((RB, C_LANES), lambda i: (i, 0)),
                  pl.BlockSpec((C_LANES, C_LANES), lambda i: (0, 0)),
                  pl.BlockSpec((RB, DW), lambda i: (i, 0)),
                  pl.BlockSpec((RB, DW), lambda i: (i, 0))],
        out_specs=[pl.BlockSpec((NC, RB, CH), lambda i: (0, i, 0)),
                   pl.BlockSpec((RB, 8), lambda i: (i, 0))],
        out_shape=[jax.ShapeDtypeStruct((NC, np_rows, CH), jnp.float32),
                   jax.ShapeDtypeStruct((np_rows, 8), jnp.float32)],
    )(x_pad, w, deg0, deg1)


def _sc_degree(dst4, np_rows):
    kb = dst4.shape[2]
    stripe = np_rows // NS
    mesh = plsc.VectorSubcoreMesh(core_axis_name="c", subcore_axis_name="s")

    @functools.partial(
        pl.kernel,
        out_type=jax.ShapeDtypeStruct((NC, np_rows, DW), jnp.float32),
        mesh=mesh,
        scratch_types=[
            pltpu.VMEM((kb, B), jnp.int32),
            pltpu.VMEM((B, DW), jnp.float32),
            pltpu.VMEM((stripe, DW), jnp.float32),
            pltpu.VMEM_SHARED((np_rows, DW), jnp.float32),
        ],
        compiler_params=pltpu.CompilerParams(use_tc_tiling_on_sc=False),
    )
    def k(dst_hbm, out_hbm, idx_v, ones_v, z_v, acc_sh):
        cid = lax.axis_index("c")
        sid = lax.axis_index("s")

        def init_ones(i, carry):
            ones_v[i, :] = jnp.full((DW,), 1.0, jnp.float32)
            return carry
        lax.fori_loop(0, B, init_ones, 0)

        def init_zeros(i, carry):
            z_v[i, :] = jnp.zeros((DW,), jnp.float32)
            return carry
        lax.fori_loop(0, stripe, init_zeros, 0)

        pltpu.sync_copy(z_v, acc_sh.at[pl.ds(sid * stripe, stripe)])
        plsc.subcore_barrier()

        pltpu.sync_copy(dst_hbm.at[cid, sid], idx_v)

        def body(bi, carry):
            pltpu.sync_copy(ones_v, acc_sh.at[idx_v.at[bi]], add=True)
            return carry
        lax.fori_loop(0, kb, body, 0)

        plsc.subcore_barrier()
        pltpu.sync_copy(acc_sh.at[pl.ds(sid * stripe, stripe)],
                        out_hbm.at[cid, pl.ds(sid * stripe, stripe)])

    return k(dst4)


def _sc_messages(g2flat, src3, dst3, np_rows):
    kb = src3.shape[1]
    stripe = np_rows // NS
    n_init = stripe // B
    mesh = plsc.VectorSubcoreMesh(core_axis_name="c", subcore_axis_name="s")

    @functools.partial(
        pl.kernel,
        out_type=jax.ShapeDtypeStruct((np_rows, C_LANES), jnp.float32),
        mesh=mesh,
        scratch_types=(
            [pltpu.VMEM((kb, B), jnp.int32),
             pltpu.VMEM((kb, B), jnp.int32)]
            + [pltpu.VMEM((B, CH), jnp.float32) for _ in range(NBUF)]
            + [pltpu.VMEM_SHARED((np_rows, CH), jnp.float32)]
            + [pltpu.SemaphoreType.DMA for _ in range(2 * NBUF)]
        ),
        compiler_params=pltpu.CompilerParams(use_tc_tiling_on_sc=False),
    )
    def k(g_hbm, src_hbm, dst_hbm, out_hbm, src_v, dst_v, *rest):
        bufs = rest[:NBUF]
        acc_sh = rest[NBUF]
        gsems = rest[NBUF + 1:NBUF + 1 + NBUF]
        ssems = rest[NBUF + 1 + NBUF:]
        cid = lax.axis_index("c")
        sid = lax.axis_index("s")

        r0 = bufs[0]
        def zrow(i, carry):
            for j in range(CH // 16):
                r0[i, pl.ds(j * 16, 16)] = jnp.zeros((16,), jnp.float32)
            return carry
        lax.fori_loop(0, B, zrow, 0)
        for t in range(n_init):
            pltpu.sync_copy(r0, acc_sh.at[pl.ds(sid * stripe + t * B, B)])
        plsc.subcore_barrier()

        pltpu.sync_copy(src_hbm.at[sid], src_v)
        pltpu.sync_copy(dst_hbm.at[sid], dst_v)

        off = (cid * np_rows).astype(jnp.int32)
        def adj(i, carry):
            for j in range(B // 16):
                sl = pl.ds(j * 16, 16)
                src_v[i, sl] = src_v[i, sl] + off
            return carry
        lax.fori_loop(0, kb, adj, 0)

        for p in range(NBUF):
            pltpu.async_copy(g_hbm.at[src_v.at[p]], bufs[p], gsems[p])

        def body(gi, carry):
            base = NBUF * gi
            for p in range(NBUF):
                pltpu.make_async_copy(
                    g_hbm.at[src_v.at[base + p]], bufs[p], gsems[p]).wait()
                pltpu.async_copy(
                    bufs[p], acc_sh.at[dst_v.at[base + p]], ssems[p],
                    add=True)

            @pl.when(gi < kb // NBUF - 1)
            def _():
                for p in range(NBUF):
                    pltpu.make_async_copy(
                        bufs[p], acc_sh.at[dst_v.at[base + p]],
                        ssems[p]).wait()
                    pltpu.async_copy(
                        g_hbm.at[src_v.at[base + NBUF + p]], bufs[p],
                        gsems[p])
            return carry
        lax.fori_loop(0, kb // NBUF, body, 0)

        for p in range(NBUF):
            pltpu.make_async_copy(
                bufs[p], acc_sh.at[dst_v.at[0]], ssems[p]).wait()

        plsc.subcore_barrier()
        pltpu.sync_copy(
            acc_sh.at[pl.ds(sid * stripe, stripe)],
            out_hbm.at[pl.ds(sid * stripe, stripe), pl.ds(cid * CH, CH)])

    return k(g2flat, src3, dst3)


def _final_body(q_ref, g_ref, dis_ref, b_ref, o_ref):
    s = dis_ref[:, :1]
    full = jnp.concatenate(
        [q_ref[:, :CH] + g_ref[0], q_ref[:, CH:] + g_ref[1]], axis=1)
    o_ref[...] = jnp.maximum(full * s + b_ref[:1], 0.0)


def _tc_final(q, g2, dis, bias):
    np_rows = g2.shape[1]
    return pl.pallas_call(
        _final_body,
        grid=(np_rows // RB,),
        in_specs=[pl.BlockSpec((RB, C_LANES), lambda i: (i, 0)),
                  pl.BlockSpec((NC, RB, CH), lambda i: (0, i, 0)),
                  pl.BlockSpec((RB, 8), lambda i: (i, 0)),
                  pl.BlockSpec((8, C_LANES), lambda i: (0, 0))],
        out_specs=pl.BlockSpec((RB, C_LANES), lambda i: (i, 0)),
        out_shape=jax.ShapeDtypeStruct((np_rows, C_LANES), jnp.float32),
    )(q, g2, dis, bias)


def kernel(x, edge_index, W, b):
    n, c = x.shape
    e = edge_index.shape[1]
    assert c == C_LANES

    align = 2048
    np_rows = -(-(n + 1) // align) * align
    chunk = NS * B * NBUF
    e_pad = -(-e // chunk) * chunk

    src = edge_index[0].astype(jnp.int32)
    dst = edge_index[1].astype(jnp.int32)
    pad_len = e_pad - e
    if pad_len:
        pad_idx = n + (jnp.arange(pad_len, dtype=jnp.int32) % (np_rows - n))
        src = jnp.concatenate([src, pad_idx])
        dst = jnp.concatenate([dst, pad_idx])
    dst4 = dst.reshape(NC, NS, -1, B)
    src3 = src.reshape(NS, -1, B)
    dst3 = dst.reshape(NS, -1, B)

    x_pad = jnp.concatenate(
        [x, jnp.zeros((np_rows - n, c), jnp.float32)], axis=0)

    deg = _sc_degree(dst4, np_rows)
    g2, dis = _tc_gscale(x_pad, W, deg[0], deg[1])
    q = _sc_messages(g2.reshape(NC * np_rows, CH), src3, dst3, np_rows)
    bias = jnp.broadcast_to(b.reshape(1, C_LANES), (8, C_LANES))
    out = _tc_final(q, g2, dis, bias)
    return out[:n]


# speedup vs baseline: 36.1768x; 1.0033x over previous
import functools

import jax
import jax.numpy as jnp
from jax import lax
from jax.experimental import pallas as pl
from jax.experimental.pallas import tpu as pltpu
from jax.experimental.pallas import tpu_sc as plsc

C_LANES = 128
CH = C_LANES // 2
NC = 2
NS = 16
B = 128
DW = 16
RB = 1024
NBUF = 4


def _gscale_body(x_ref, w_ref, d0_ref, d1_ref, g2_ref, dis_ref):
    h = jnp.dot(x_ref[...], w_ref[...], preferred_element_type=jnp.float32)
    deg = d0_ref[:, :1] + d1_ref[:, :1] + 1.0
    dis = lax.rsqrt(deg)
    g2_ref[0] = h[:, :CH] * dis
    g2_ref[1] = h[:, CH:] * dis
    dis_ref[...] = jnp.broadcast_to(dis, dis_ref.shape)


def _tc_gscale(x_pad, w, deg0, deg1):
    np_rows = x_pad.shape[0]
    return pl.pallas_call(
        _gscale_body,
        grid=(np_rows // RB,),
        in_specs=[pl.BlockSpec((RB, C_LANES), lambda i: (i, 0)),
                  pl.BlockSpec((C_LANES, C_LANES), lambda i: (0, 0)),
                  pl.BlockSpec((RB, DW), lambda i: (i, 0)),
                  pl.BlockSpec((RB, DW), lambda i: (i, 0))],
        out_specs=[pl.BlockSpec((NC, RB, CH), lambda i: (0, i, 0)),
                   pl.BlockSpec((RB, 8), lambda i: (i, 0))],
        out_shape=[jax.ShapeDtypeStruct((NC, np_rows, CH), jnp.float32),
                   jax.ShapeDtypeStruct((np_rows, 8), jnp.float32)],
    )(x_pad, w, deg0, deg1)


def _sc_degree(dst4, np_rows):
    kb = dst4.shape[2]
    stripe = np_rows // NS
    mesh = plsc.VectorSubcoreMesh(core_axis_name="c", subcore_axis_name="s")

    @functools.partial(
        pl.kernel,
        out_type=jax.ShapeDtypeStruct((NC, np_rows, DW), jnp.float32),
        mesh=mesh,
        scratch_types=[
            pltpu.VMEM((kb, B), jnp.int32),
            pltpu.VMEM((B, DW), jnp.float32),
            pltpu.VMEM((stripe, DW), jnp.float32),
            pltpu.VMEM_SHARED((np_rows, DW), jnp.float32),
            pltpu.SemaphoreType.DMA,
        ],
        compiler_params=pltpu.CompilerParams(use_tc_tiling_on_sc=False),
    )
    def k(dst_hbm, out_hbm, idx_v, ones_v, z_v, acc_sh, ssem):
        cid = lax.axis_index("c")
        sid = lax.axis_index("s")

        def init_ones(i, carry):
            ones_v[i, :] = jnp.full((DW,), 1.0, jnp.float32)
            return carry
        lax.fori_loop(0, B, init_ones, 0)

        def init_zeros(i, carry):
            z_v[i, :] = jnp.zeros((DW,), jnp.float32)
            return carry
        lax.fori_loop(0, stripe, init_zeros, 0)

        pltpu.sync_copy(z_v, acc_sh.at[pl.ds(sid * stripe, stripe)])
        plsc.subcore_barrier()

        pltpu.sync_copy(dst_hbm.at[cid, sid], idx_v)

        fire = next(f for f in (8, 4, 2, 1) if kb % f == 0)
        def body(bi, carry):
            base = fire * bi
            for p in range(fire):
                pltpu.async_copy(ones_v, acc_sh.at[idx_v.at[base + p]],
                                 ssem, add=True)
            for p in range(fire):
                pltpu.make_async_copy(ones_v, acc_sh.at[idx_v.at[base]],
                                      ssem).wait()
            return carry
        lax.fori_loop(0, kb // fire, body, 0)

        plsc.subcore_barrier()
        pltpu.sync_copy(acc_sh.at[pl.ds(sid * stripe, stripe)],
                        out_hbm.at[cid, pl.ds(sid * stripe, stripe)])

    return k(dst4)


def _sc_messages(g2flat, src3, dst3, np_rows):
    kb = src3.shape[1]
    stripe = np_rows // NS
    n_init = stripe // B
    mesh = plsc.VectorSubcoreMesh(core_axis_name="c", subcore_axis_name="s")

    @functools.partial(
        pl.kernel,
        out_type=jax.ShapeDtypeStruct((np_rows, C_LANES), jnp.float32),
        mesh=mesh,
        scratch_types=(
            [pltpu.VMEM((kb, B), jnp.int32),
             pltpu.VMEM((kb, B), jnp.int32)]
            + [pltpu.VMEM((B, CH), jnp.float32) for _ in range(NBUF)]
            + [pltpu.VMEM_SHARED((np_rows, CH), jnp.float32)]
            + [pltpu.SemaphoreType.DMA for _ in range(2 * NBUF)]
        ),
        compiler_params=pltpu.CompilerParams(use_tc_tiling_on_sc=False),
    )
    def k(g_hbm, src_hbm, dst_hbm, out_hbm, src_v, dst_v, *rest):
        bufs = rest[:NBUF]
        acc_sh = rest[NBUF]
        gsems = rest[NBUF + 1:NBUF + 1 + NBUF]
        ssems = rest[NBUF + 1 + NBUF:]
        cid = lax.axis_index("c")
        sid = lax.axis_index("s")

        r0 = bufs[0]
        def zrow(i, carry):
            for j in range(CH // 16):
                r0[i, pl.ds(j * 16, 16)] = jnp.zeros((16,), jnp.float32)
            return carry
        lax.fori_loop(0, B, zrow, 0)
        for t in range(n_init):
            pltpu.sync_copy(r0, acc_sh.at[pl.ds(sid * stripe + t * B, B)])
        plsc.subcore_barrier()

        pltpu.sync_copy(src_hbm.at[sid], src_v)
        pltpu.sync_copy(dst_hbm.at[sid], dst_v)

        off = (cid * np_rows).astype(jnp.int32)
        def adj(i, carry):
            for j in range(B // 16):
                sl = pl.ds(j * 16, 16)
                src_v[i, sl] = src_v[i, sl] + off
            return carry
        lax.fori_loop(0, kb, adj, 0)

        for p in range(NBUF):
            pltpu.async_copy(g_hbm.at[src_v.at[p]], bufs[p], gsems[p])

        def body(gi, carry):
            base = NBUF * gi
            for p in range(NBUF):
                pltpu.make_async_copy(
                    g_hbm.at[src_v.at[base + p]], bufs[p], gsems[p]).wait()
                pltpu.async_copy(
                    bufs[p], acc_sh.at[dst_v.at[base + p]], ssems[p],
                    add=True)

            @pl.when(gi < kb // NBUF - 1)
            def _():
                for p in range(NBUF):
                    pltpu.make_async_copy(
                        bufs[p], acc_sh.at[dst_v.at[base + p]],
                        ssems[p]).wait()
                    pltpu.async_copy(
                        g_hbm.at[src_v.at[base + NBUF + p]], bufs[p],
                        gsems[p])
            return carry
        lax.fori_loop(0, kb // NBUF, body, 0)

        for p in range(NBUF):
            pltpu.make_async_copy(
                bufs[p], acc_sh.at[dst_v.at[0]], ssems[p]).wait()

        plsc.subcore_barrier()
        pltpu.sync_copy(
            acc_sh.at[pl.ds(sid * stripe, stripe)],
            out_hbm.at[pl.ds(sid * stripe, stripe), pl.ds(cid * CH, CH)])

    return k(g2flat, src3, dst3)


def _final_body(q_ref, g_ref, dis_ref, b_ref, o_ref):
    s = dis_ref[:, :1]
    full = jnp.concatenate(
        [q_ref[:, :CH] + g_ref[0], q_ref[:, CH:] + g_ref[1]], axis=1)
    o_ref[...] = jnp.maximum(full * s + b_ref[:1], 0.0)


def _tc_final(q, g2, dis, bias, n):
    rb_f = max(r for r in (2048, 1024, 512, 400, 256, 128, 16, 8)
               if n % r == 0)
    return pl.pallas_call(
        _final_body,
        grid=(n // rb_f,),
        in_specs=[pl.BlockSpec((rb_f, C_LANES), lambda i: (i, 0)),
                  pl.BlockSpec((NC, rb_f, CH), lambda i: (0, i, 0)),
                  pl.BlockSpec((rb_f, 8), lambda i: (i, 0)),
                  pl.BlockSpec((8, C_LANES), lambda i: (0, 0))],
        out_specs=pl.BlockSpec((rb_f, C_LANES), lambda i: (i, 0)),
        out_shape=jax.ShapeDtypeStruct((n, C_LANES), jnp.float32),
    )(q, g2, dis, bias)


def kernel(x, edge_index, W, b):
    n, c = x.shape
    e = edge_index.shape[1]
    assert c == C_LANES

    align = 2048
    np_rows = -(-(n + 1) // align) * align
    chunk = NS * B * NBUF
    e_pad = -(-e // chunk) * chunk

    src = edge_index[0].astype(jnp.int32)
    dst = edge_index[1].astype(jnp.int32)
    pad_len = e_pad - e
    if pad_len:
        pad_idx = n + (jnp.arange(pad_len, dtype=jnp.int32) % (np_rows - n))
        src = jnp.concatenate([src, pad_idx])
        dst = jnp.concatenate([dst, pad_idx])
    dst4 = dst.reshape(NC, NS, -1, B)
    src3 = src.reshape(NS, -1, B)
    dst3 = dst.reshape(NS, -1, B)

    x_pad = jnp.concatenate(
        [x, jnp.zeros((np_rows - n, c), jnp.float32)], axis=0)

    deg = _sc_degree(dst4, np_rows)
    g2, dis = _tc_gscale(x_pad, W, deg[0], deg[1])
    q = _sc_messages(g2.reshape(NC * np_rows, CH), src3, dst3, np_rows)
    bias = jnp.broadcast_to(b.reshape(1, C_LANES), (8, C_LANES))
    return _tc_final(q, g2, dis, bias, n)


# speedup vs baseline: 40.1038x; 1.1086x over previous
import functools

import jax
import jax.numpy as jnp
from jax import lax
from jax.experimental import pallas as pl
from jax.experimental.pallas import tpu as pltpu
from jax.experimental.pallas import tpu_sc as plsc

C_LANES = 128
CH = C_LANES // 2
NC = 2
NS = 16
B = 128
DW = 16
RB = 1024
NBUF = 4


def _gscale_body(x_ref, w_ref, d0_ref, d1_ref, g2_ref, dis_ref):
    h = jnp.dot(x_ref[...], w_ref[...], preferred_element_type=jnp.float32)
    deg = d0_ref[:, :1] + d1_ref[:, :1] + 1.0
    dis = lax.rsqrt(deg)
    g2_ref[0] = h[:, :CH] * dis
    g2_ref[1] = h[:, CH:] * dis
    dis_ref[...] = jnp.broadcast_to(dis, dis_ref.shape)


def _tc_gscale(x, w, deg0, deg1, np_rows):
    return pl.pallas_call(
        _gscale_body,
        grid=(np_rows // RB,),
        in_specs=[pl.BlockSpec((RB, C_LANES), lambda i: (i, 0)),
                  pl.BlockSpec((C_LANES, C_LANES), lambda i: (0, 0)),
                  pl.BlockSpec((RB, DW), lambda i: (i, 0)),
                  pl.BlockSpec((RB, DW), lambda i: (i, 0))],
        out_specs=[pl.BlockSpec((NC, RB, CH), lambda i: (0, i, 0)),
                   pl.BlockSpec((RB, 8), lambda i: (i, 0))],
        out_shape=[jax.ShapeDtypeStruct((NC, np_rows, CH), jnp.float32),
                   jax.ShapeDtypeStruct((np_rows, 8), jnp.float32)],
    )(x, w, deg0, deg1)


def _sc_degree(ei3, np_rows):
    tb = ei3.shape[1]
    tbc = tb // NC
    q, r = divmod(tbc, NS)
    kb_max = q + (1 if r else 0)
    stripe = np_rows // NS
    mesh = plsc.VectorSubcoreMesh(core_axis_name="c", subcore_axis_name="s")

    @functools.partial(
        pl.kernel,
        out_type=jax.ShapeDtypeStruct((NC, np_rows, DW), jnp.float32),
        mesh=mesh,
        scratch_types=[
            pltpu.VMEM((kb_max, B), jnp.int32),
            pltpu.VMEM((B, DW), jnp.float32),
            pltpu.VMEM((stripe, DW), jnp.float32),
            pltpu.VMEM_SHARED((np_rows, DW), jnp.float32),
            pltpu.SemaphoreType.DMA,
        ],
        compiler_params=pltpu.CompilerParams(use_tc_tiling_on_sc=False),
    )
    def k(ei_hbm, out_hbm, idx_v, ones_v, z_v, acc_sh, ssem):
        cid = lax.axis_index("c")
        sid = lax.axis_index("s")

        def init_ones(i, carry):
            ones_v[i, :] = jnp.full((DW,), 1.0, jnp.float32)
            return carry
        lax.fori_loop(0, B, init_ones, 0)

        def init_zeros(i, carry):
            z_v[i, :] = jnp.zeros((DW,), jnp.float32)
            return carry
        lax.fori_loop(0, stripe, init_zeros, 0)

        pltpu.sync_copy(z_v, acc_sh.at[pl.ds(sid * stripe, stripe)])
        plsc.subcore_barrier()

        start = cid * tbc + q * sid + jnp.minimum(sid, r)
        kb_dyn = q + jnp.where(sid < r, 1, 0)
        pltpu.sync_copy(ei_hbm.at[1, pl.ds(start, q)], idx_v.at[pl.ds(0, q)])
        if r:
            @pl.when(sid < r)
            def _():
                pltpu.sync_copy(ei_hbm.at[1, pl.ds(start + q, 1)],
                                idx_v.at[pl.ds(q, 1)])

        fire = 8
        nfull = kb_dyn // fire
        def body(bi, carry):
            base = fire * bi
            for p in range(fire):
                pltpu.async_copy(ones_v, acc_sh.at[idx_v.at[base + p]],
                                 ssem, add=True)
            for p in range(fire):
                pltpu.make_async_copy(ones_v, acc_sh.at[idx_v.at[base]],
                                      ssem).wait()
            return carry
        lax.fori_loop(0, nfull, body, 0)

        def tail(bi, carry):
            pltpu.sync_copy(ones_v, acc_sh.at[idx_v.at[bi]], add=True)
            return carry
        lax.fori_loop(nfull * fire, kb_dyn, tail, 0)

        plsc.subcore_barrier()
        pltpu.sync_copy(acc_sh.at[pl.ds(sid * stripe, stripe)],
                        out_hbm.at[cid, pl.ds(sid * stripe, stripe)])

    return k(ei3)


def _sc_messages(g2flat, ei3, np_rows):
    tb = ei3.shape[1]
    q, r = divmod(tb, NS)
    kb_max = q + (1 if r else 0)
    stripe = np_rows // NS
    n_init = stripe // B
    mesh = plsc.VectorSubcoreMesh(core_axis_name="c", subcore_axis_name="s")

    @functools.partial(
        pl.kernel,
        out_type=jax.ShapeDtypeStruct((np_rows, C_LANES), jnp.float32),
        mesh=mesh,
        scratch_types=(
            [pltpu.VMEM((kb_max, B), jnp.int32),
             pltpu.VMEM((kb_max, B), jnp.int32)]
            + [pltpu.VMEM((B, CH), jnp.float32) for _ in range(NBUF)]
            + [pltpu.VMEM_SHARED((np_rows, CH), jnp.float32)]
            + [pltpu.SemaphoreType.DMA for _ in range(2 * NBUF)]
        ),
        compiler_params=pltpu.CompilerParams(use_tc_tiling_on_sc=False),
    )
    def k(g_hbm, ei_hbm, out_hbm, src_v, dst_v, *rest):
        bufs = rest[:NBUF]
        acc_sh = rest[NBUF]
        gsems = rest[NBUF + 1:NBUF + 1 + NBUF]
        ssems = rest[NBUF + 1 + NBUF:]
        cid = lax.axis_index("c")
        sid = lax.axis_index("s")

        r0 = bufs[0]
        def zrow(i, carry):
            for j in range(CH // 16):
                r0[i, pl.ds(j * 16, 16)] = jnp.zeros((16,), jnp.float32)
            return carry
        lax.fori_loop(0, B, zrow, 0)
        for t in range(n_init):
            pltpu.sync_copy(r0, acc_sh.at[pl.ds(sid * stripe + t * B, B)])
        plsc.subcore_barrier()

        start = q * sid + jnp.minimum(sid, r)
        kb_dyn = q + jnp.where(sid < r, 1, 0)
        pltpu.sync_copy(ei_hbm.at[0, pl.ds(start, q)], src_v.at[pl.ds(0, q)])
        pltpu.sync_copy(ei_hbm.at[1, pl.ds(start, q)], dst_v.at[pl.ds(0, q)])
        if r:
            @pl.when(sid < r)
            def _():
                pltpu.sync_copy(ei_hbm.at[0, pl.ds(start + q, 1)],
                                src_v.at[pl.ds(q, 1)])
                pltpu.sync_copy(ei_hbm.at[1, pl.ds(start + q, 1)],
                                dst_v.at[pl.ds(q, 1)])

        off = (cid * np_rows).astype(jnp.int32)
        def adj(i, carry):
            for j in range(B // 16):
                sl = pl.ds(j * 16, 16)
                src_v[i, sl] = src_v[i, sl] + off
            return carry
        lax.fori_loop(0, kb_max, adj, 0)

        nfull = kb_dyn // NBUF
        for p in range(NBUF):
            pltpu.async_copy(g_hbm.at[src_v.at[p]], bufs[p], gsems[p])

        def body(gi, carry):
            base = NBUF * gi
            for p in range(NBUF):
                pltpu.make_async_copy(
                    g_hbm.at[src_v.at[base + p]], bufs[p], gsems[p]).wait()
                pltpu.async_copy(
                    bufs[p], acc_sh.at[dst_v.at[base + p]], ssems[p],
                    add=True)

            @pl.when(gi < nfull - 1)
            def _():
                for p in range(NBUF):
                    pltpu.make_async_copy(
                        bufs[p], acc_sh.at[dst_v.at[base + p]],
                        ssems[p]).wait()
                    pltpu.async_copy(
                        g_hbm.at[src_v.at[base + NBUF + p]], bufs[p],
                        gsems[p])
            return carry
        lax.fori_loop(0, nfull, body, 0)

        for p in range(NBUF):
            pltpu.make_async_copy(
                bufs[p], acc_sh.at[dst_v.at[0]], ssems[p]).wait()

        def tail(bi, carry):
            pltpu.async_copy(g_hbm.at[src_v.at[bi]], bufs[0], gsems[0])
            pltpu.make_async_copy(
                g_hbm.at[src_v.at[bi]], bufs[0], gsems[0]).wait()
            pltpu.sync_copy(bufs[0], acc_sh.at[dst_v.at[bi]], add=True)
            return carry
        lax.fori_loop(nfull * NBUF, kb_dyn, tail, 0)

        plsc.subcore_barrier()
        pltpu.sync_copy(
            acc_sh.at[pl.ds(sid * stripe, stripe)],
            out_hbm.at[pl.ds(sid * stripe, stripe), pl.ds(cid * CH, CH)])

    return k(g2flat, ei3)


def _final_body(q_ref, g_ref, dis_ref, b_ref, o_ref):
    s = dis_ref[:, :1]
    full = jnp.concatenate(
        [q_ref[:, :CH] + g_ref[0], q_ref[:, CH:] + g_ref[1]], axis=1)
    o_ref[...] = jnp.maximum(full * s + b_ref[:1], 0.0)


def _tc_final(q, g2, dis, bias, n):
    rb_f = max(r for r in (2048, 2000, 1024, 512, 400, 256, 128, 16, 8)
               if n % r == 0)
    return pl.pallas_call(
        _final_body,
        grid=(n // rb_f,),
        in_specs=[pl.BlockSpec((rb_f, C_LANES), lambda i: (i, 0)),
                  pl.BlockSpec((NC, rb_f, CH), lambda i: (0, i, 0)),
                  pl.BlockSpec((rb_f, 8), lambda i: (i, 0)),
                  pl.BlockSpec((8, C_LANES), lambda i: (0, 0))],
        out_specs=pl.BlockSpec((rb_f, C_LANES), lambda i: (i, 0)),
        out_shape=jax.ShapeDtypeStruct((n, C_LANES), jnp.float32),
    )(q, g2, dis, bias)


def kernel(x, edge_index, W, b):
    n, c = x.shape
    e = edge_index.shape[1]
    assert c == C_LANES

    align = 2048
    np_rows = -(-(n + 1) // align) * align
    assert e % (NC * B) == 0, "edge count must be a multiple of 256"
    ei3 = edge_index.astype(jnp.int32).reshape(2, e // B, B)

    deg = _sc_degree(ei3, np_rows)
    g2, dis = _tc_gscale(x, W, deg[0], deg[1], np_rows)
    q = _sc_messages(g2.reshape(NC * np_rows, CH), ei3, np_rows)
    bias = jnp.broadcast_to(b.reshape(1, C_LANES), (8, C_LANES))
    return _tc_final(q, g2, dis, bias, n)


# speedup vs baseline: 40.2203x; 1.0029x over previous
import functools

import jax
import jax.numpy as jnp
from jax import lax
from jax.experimental import pallas as pl
from jax.experimental.pallas import tpu as pltpu
from jax.experimental.pallas import tpu_sc as plsc

C_LANES = 128
CH = C_LANES // 2
NC = 2
NS = 16
B = 128
DW = 16
RB = 1024
NBUF = 4


def _gscale_body(x_ref, w_ref, d0_ref, d1_ref, g2_ref, dis_ref):
    h = jnp.dot(x_ref[...], w_ref[...], preferred_element_type=jnp.float32)
    deg = d0_ref[:, :1] + d1_ref[:, :1] + 1.0
    dis = lax.rsqrt(deg)
    g2_ref[0] = h[:, :CH] * dis
    g2_ref[1] = h[:, CH:] * dis
    dis_ref[...] = jnp.broadcast_to(dis, dis_ref.shape)


def _tc_gscale(x, w, deg0, deg1, np_rows):
    return pl.pallas_call(
        _gscale_body,
        grid=(np_rows // RB,),
        in_specs=[pl.BlockSpec((RB, C_LANES), lambda i: (i, 0)),
                  pl.BlockSpec((C_LANES, C_LANES), lambda i: (0, 0)),
                  pl.BlockSpec((RB, DW), lambda i: (i, 0)),
                  pl.BlockSpec((RB, DW), lambda i: (i, 0))],
        out_specs=[pl.BlockSpec((NC, RB, CH), lambda i: (0, i, 0)),
                   pl.BlockSpec((RB, 8), lambda i: (i, 0))],
        out_shape=[jax.ShapeDtypeStruct((NC, np_rows, CH), jnp.float32),
                   jax.ShapeDtypeStruct((np_rows, 8), jnp.float32)],
    )(x, w, deg0, deg1)


def _sc_degree(ei3, np_rows):
    tb = ei3.shape[1]
    tbc = tb // NC
    q, r = divmod(tbc, NS)
    kb_max = q + (1 if r else 0)
    stripe = np_rows // NS
    mesh = plsc.VectorSubcoreMesh(core_axis_name="c", subcore_axis_name="s")

    @functools.partial(
        pl.kernel,
        out_type=jax.ShapeDtypeStruct((NC, np_rows, DW), jnp.float32),
        mesh=mesh,
        scratch_types=[
            pltpu.VMEM((kb_max, B), jnp.int32),
            pltpu.VMEM((B, DW), jnp.float32),
            pltpu.VMEM((stripe, DW), jnp.float32),
            pltpu.VMEM_SHARED((np_rows, DW), jnp.float32),
            pltpu.SemaphoreType.DMA,
        ],
        compiler_params=pltpu.CompilerParams(use_tc_tiling_on_sc=False),
    )
    def k(ei_hbm, out_hbm, idx_v, ones_v, z_v, acc_sh, ssem):
        cid = lax.axis_index("c")
        sid = lax.axis_index("s")

        def init_ones(i, carry):
            ones_v[i, :] = jnp.full((DW,), 1.0, jnp.float32)
            return carry
        lax.fori_loop(0, B, init_ones, 0)

        def init_zeros(i, carry):
            z_v[i, :] = jnp.zeros((DW,), jnp.float32)
            return carry
        lax.fori_loop(0, stripe, init_zeros, 0)

        pltpu.sync_copy(z_v, acc_sh.at[pl.ds(sid * stripe, stripe)])
        plsc.subcore_barrier()

        start = cid * tbc + q * sid + jnp.minimum(sid, r)
        kb_dyn = q + jnp.where(sid < r, 1, 0)
        pltpu.sync_copy(ei_hbm.at[1, pl.ds(start, q)], idx_v.at[pl.ds(0, q)])
        if r:
            @pl.when(sid < r)
            def _():
                pltpu.sync_copy(ei_hbm.at[1, pl.ds(start + q, 1)],
                                idx_v.at[pl.ds(q, 1)])

        fire = 8
        nfull = kb_dyn // fire
        def body(bi, carry):
            base = fire * bi
            for p in range(fire):
                pltpu.async_copy(ones_v, acc_sh.at[idx_v.at[base + p]],
                                 ssem, add=True)
            for p in range(fire):
                pltpu.make_async_copy(ones_v, acc_sh.at[idx_v.at[base]],
                                      ssem).wait()
            return carry
        lax.fori_loop(0, nfull, body, 0)

        def tail(bi, carry):
            pltpu.sync_copy(ones_v, acc_sh.at[idx_v.at[bi]], add=True)
            return carry
        lax.fori_loop(nfull * fire, kb_dyn, tail, 0)

        plsc.subcore_barrier()
        pltpu.sync_copy(acc_sh.at[pl.ds(sid * stripe, stripe)],
                        out_hbm.at[cid, pl.ds(sid * stripe, stripe)])

    return k(ei3)


def _sc_messages(g2, ei3, np_rows):
    tb = ei3.shape[1]
    q, r = divmod(tb, NS)
    kb_max = q + (1 if r else 0)
    stripe = np_rows // NS
    n_init = stripe // B
    mesh = plsc.VectorSubcoreMesh(core_axis_name="c", subcore_axis_name="s")

    @functools.partial(
        pl.kernel,
        out_type=jax.ShapeDtypeStruct((np_rows, C_LANES), jnp.float32),
        mesh=mesh,
        scratch_types=(
            [pltpu.VMEM((kb_max, B), jnp.int32),
             pltpu.VMEM((kb_max, B), jnp.int32)]
            + [pltpu.VMEM((B, CH), jnp.float32) for _ in range(NBUF)]
            + [pltpu.VMEM_SHARED((np_rows, CH), jnp.float32)]
            + [pltpu.SemaphoreType.DMA for _ in range(2 * NBUF)]
        ),
        compiler_params=pltpu.CompilerParams(use_tc_tiling_on_sc=False),
    )
    def k(g_hbm, ei_hbm, out_hbm, src_v, dst_v, *rest):
        bufs = rest[:NBUF]
        acc_sh = rest[NBUF]
        gsems = rest[NBUF + 1:NBUF + 1 + NBUF]
        ssems = rest[NBUF + 1 + NBUF:]
        cid = lax.axis_index("c")
        sid = lax.axis_index("s")

        r0 = bufs[0]
        def zrow(i, carry):
            for j in range(CH // 16):
                r0[i, pl.ds(j * 16, 16)] = jnp.zeros((16,), jnp.float32)
            return carry
        lax.fori_loop(0, B, zrow, 0)
        for t in range(n_init):
            pltpu.sync_copy(r0, acc_sh.at[pl.ds(sid * stripe + t * B, B)])
        plsc.subcore_barrier()

        start = q * sid + jnp.minimum(sid, r)
        kb_dyn = q + jnp.where(sid < r, 1, 0)
        pltpu.sync_copy(ei_hbm.at[0, pl.ds(start, q)], src_v.at[pl.ds(0, q)])
        pltpu.sync_copy(ei_hbm.at[1, pl.ds(start, q)], dst_v.at[pl.ds(0, q)])
        if r:
            @pl.when(sid < r)
            def _():
                pltpu.sync_copy(ei_hbm.at[0, pl.ds(start + q, 1)],
                                src_v.at[pl.ds(q, 1)])
                pltpu.sync_copy(ei_hbm.at[1, pl.ds(start + q, 1)],
                                dst_v.at[pl.ds(q, 1)])

        g_core = g_hbm.at[cid]

        nfull = kb_dyn // NBUF
        for p in range(NBUF):
            pltpu.async_copy(g_core.at[src_v.at[p]], bufs[p], gsems[p])

        def body(gi, carry):
            base = NBUF * gi
            for p in range(NBUF):
                pltpu.make_async_copy(
                    g_core.at[src_v.at[base + p]], bufs[p], gsems[p]).wait()
                pltpu.async_copy(
                    bufs[p], acc_sh.at[dst_v.at[base + p]], ssems[p],
                    add=True)

            @pl.when(gi < nfull - 1)
            def _():
                for p in range(NBUF):
                    pltpu.make_async_copy(
                        bufs[p], acc_sh.at[dst_v.at[base + p]],
                        ssems[p]).wait()
                    pltpu.async_copy(
                        g_core.at[src_v.at[base + NBUF + p]], bufs[p],
                        gsems[p])
            return carry
        lax.fori_loop(0, nfull, body, 0)

        for p in range(NBUF):
            pltpu.make_async_copy(
                bufs[p], acc_sh.at[dst_v.at[0]], ssems[p]).wait()

        def tail(bi, carry):
            pltpu.async_copy(g_core.at[src_v.at[bi]], bufs[0], gsems[0])
            pltpu.make_async_copy(
                g_core.at[src_v.at[bi]], bufs[0], gsems[0]).wait()
            pltpu.sync_copy(bufs[0], acc_sh.at[dst_v.at[bi]], add=True)
            return carry
        lax.fori_loop(nfull * NBUF, kb_dyn, tail, 0)

        plsc.subcore_barrier()
        pltpu.sync_copy(
            acc_sh.at[pl.ds(sid * stripe, stripe)],
            out_hbm.at[pl.ds(sid * stripe, stripe), pl.ds(cid * CH, CH)])

    return k(g2, ei3)


def _final_body(q_ref, g_ref, dis_ref, b_ref, o_ref):
    s = dis_ref[:, :1]
    full = jnp.concatenate(
        [q_ref[:, :CH] + g_ref[0], q_ref[:, CH:] + g_ref[1]], axis=1)
    o_ref[...] = jnp.maximum(full * s + b_ref[:1], 0.0)


def _tc_final(q, g2, dis, bias, n):
    rb_f = max(r for r in (2048, 2000, 1024, 512, 400, 256, 128, 16, 8)
               if n % r == 0)
    return pl.pallas_call(
        _final_body,
        grid=(n // rb_f,),
        in_specs=[pl.BlockSpec((rb_f, C_LANES), lambda i: (i, 0)),
                  pl.BlockSpec((NC, rb_f, CH), lambda i: (0, i, 0)),
                  pl.BlockSpec((rb_f, 8), lambda i: (i, 0)),
                  pl.BlockSpec((8, C_LANES), lambda i: (0, 0))],
        out_specs=pl.BlockSpec((rb_f, C_LANES), lambda i: (i, 0)),
        out_shape=jax.ShapeDtypeStruct((n, C_LANES), jnp.float32),
    )(q, g2, dis, bias)


def kernel(x, edge_index, W, b):
    n, c = x.shape
    e = edge_index.shape[1]
    assert c == C_LANES

    align = 2048
    np_rows = -(-(n + 1) // align) * align
    assert e % (NC * B) == 0, "edge count must be a multiple of 256"
    ei3 = edge_index.astype(jnp.int32).reshape(2, e // B, B)

    deg = _sc_degree(ei3, np_rows)
    g2, dis = _tc_gscale(x, W, deg[0], deg[1], np_rows)
    q = _sc_messages(g2, ei3, np_rows)
    bias = jnp.broadcast_to(b.reshape(1, C_LANES), (8, C_LANES))
    return _tc_final(q, g2, dis, bias, n)


# speedup vs baseline: 42.3843x; 1.0538x over previous
import functools

import jax
import jax.numpy as jnp
from jax import lax
from jax.experimental import pallas as pl
from jax.experimental.pallas import tpu as pltpu
from jax.experimental.pallas import tpu_sc as plsc

C_LANES = 128
CH = C_LANES // 2
NC = 2
NS = 16
B = 128
DW = 8
RB = 1024
NBUF = 4


def _gscale_body(x_ref, w_ref, d_ref, g2_ref, dis_ref):
    h = jnp.dot(x_ref[...], w_ref[...], preferred_element_type=jnp.float32)
    deg = d_ref[0, :, :1] + d_ref[1, :, :1] + 1.0
    dis = lax.rsqrt(deg)
    g2_ref[0] = h[:, :CH] * dis
    g2_ref[1] = h[:, CH:] * dis
    dis_ref[...] = jnp.broadcast_to(dis, dis_ref.shape)


def _tc_gscale(x, w, deg, np_rows):
    return pl.pallas_call(
        _gscale_body,
        grid=(np_rows // RB,),
        in_specs=[pl.BlockSpec((RB, C_LANES), lambda i: (i, 0)),
                  pl.BlockSpec((C_LANES, C_LANES), lambda i: (0, 0)),
                  pl.BlockSpec((NC, RB, DW), lambda i: (0, i, 0))],
        out_specs=[pl.BlockSpec((NC, RB, CH), lambda i: (0, i, 0)),
                   pl.BlockSpec((RB, 8), lambda i: (i, 0))],
        out_shape=[jax.ShapeDtypeStruct((NC, np_rows, CH), jnp.float32),
                   jax.ShapeDtypeStruct((np_rows, 8), jnp.float32)],
    )(x, w, deg)


def _sc_degree(ei3, np_rows):
    tb = ei3.shape[1]
    tbc = tb // NC
    q, r = divmod(tbc, NS)
    kb_max = q + (1 if r else 0)
    stripe = np_rows // NS
    mesh = plsc.VectorSubcoreMesh(core_axis_name="c", subcore_axis_name="s")

    @functools.partial(
        pl.kernel,
        out_type=jax.ShapeDtypeStruct((NC, np_rows, DW), jnp.float32),
        mesh=mesh,
        scratch_types=[
            pltpu.VMEM((kb_max, B), jnp.int32),
            pltpu.VMEM((B, DW), jnp.float32),
            pltpu.VMEM_SHARED((np_rows, DW), jnp.float32),
            pltpu.SemaphoreType.DMA,
        ],
        compiler_params=pltpu.CompilerParams(use_tc_tiling_on_sc=False),
    )
    def k(ei_hbm, ones_hbm, zeros_hbm, out_hbm, idx_v, ones_v, acc_sh, ssem):
        cid = lax.axis_index("c")
        sid = lax.axis_index("s")

        pltpu.sync_copy(ones_hbm, ones_v)
        pltpu.sync_copy(zeros_hbm, acc_sh.at[pl.ds(sid * stripe, stripe)])
        plsc.subcore_barrier()

        start = cid * tbc + q * sid + jnp.minimum(sid, r)
        kb_dyn = q + jnp.where(sid < r, 1, 0)
        pltpu.sync_copy(ei_hbm.at[1, pl.ds(start, q)], idx_v.at[pl.ds(0, q)])
        if r:
            @pl.when(sid < r)
            def _():
                pltpu.sync_copy(ei_hbm.at[1, pl.ds(start + q, 1)],
                                idx_v.at[pl.ds(q, 1)])

        fire = 8
        nfull = kb_dyn // fire
        def body(bi, carry):
            base = fire * bi
            for p in range(fire):
                pltpu.async_copy(ones_v, acc_sh.at[idx_v.at[base + p]],
                                 ssem, add=True)
            for p in range(fire):
                pltpu.make_async_copy(ones_v, acc_sh.at[idx_v.at[base]],
                                      ssem).wait()
            return carry
        lax.fori_loop(0, nfull, body, 0)

        def tail(bi, carry):
            pltpu.sync_copy(ones_v, acc_sh.at[idx_v.at[bi]], add=True)
            return carry
        lax.fori_loop(nfull * fire, kb_dyn, tail, 0)

        plsc.subcore_barrier()
        pltpu.sync_copy(acc_sh.at[pl.ds(sid * stripe, stripe)],
                        out_hbm.at[cid, pl.ds(sid * stripe, stripe)])

    return k(ei3, jnp.ones((B, DW), jnp.float32),
             jnp.zeros((stripe, DW), jnp.float32))


def _sc_messages(g2, ei3, np_rows):
    tb = ei3.shape[1]
    q, r = divmod(tb, NS)
    kb_max = q + (1 if r else 0)
    stripe = np_rows // NS
    n_init = stripe // B
    mesh = plsc.VectorSubcoreMesh(core_axis_name="c", subcore_axis_name="s")

    @functools.partial(
        pl.kernel,
        out_type=jax.ShapeDtypeStruct((np_rows, C_LANES), jnp.float32),
        mesh=mesh,
        scratch_types=(
            [pltpu.VMEM((kb_max, B), jnp.int32),
             pltpu.VMEM((kb_max, B), jnp.int32)]
            + [pltpu.VMEM((B, CH), jnp.float32) for _ in range(NBUF)]
            + [pltpu.VMEM_SHARED((np_rows, CH), jnp.float32)]
            + [pltpu.SemaphoreType.DMA for _ in range(2 * NBUF)]
        ),
        compiler_params=pltpu.CompilerParams(use_tc_tiling_on_sc=False),
    )
    def k(g_hbm, ei_hbm, out_hbm, src_v, dst_v, *rest):
        bufs = rest[:NBUF]
        acc_sh = rest[NBUF]
        gsems = rest[NBUF + 1:NBUF + 1 + NBUF]
        ssems = rest[NBUF + 1 + NBUF:]
        cid = lax.axis_index("c")
        sid = lax.axis_index("s")

        r0 = bufs[0]
        def zrow(i, carry):
            for j in range(CH // 16):
                r0[i, pl.ds(j * 16, 16)] = jnp.zeros((16,), jnp.float32)
            return carry
        lax.fori_loop(0, B, zrow, 0)
        for t in range(n_init):
            pltpu.sync_copy(r0, acc_sh.at[pl.ds(sid * stripe + t * B, B)])
        plsc.subcore_barrier()

        start = q * sid + jnp.minimum(sid, r)
        kb_dyn = q + jnp.where(sid < r, 1, 0)
        pltpu.sync_copy(ei_hbm.at[0, pl.ds(start, q)], src_v.at[pl.ds(0, q)])
        pltpu.sync_copy(ei_hbm.at[1, pl.ds(start, q)], dst_v.at[pl.ds(0, q)])
        if r:
            @pl.when(sid < r)
            def _():
                pltpu.sync_copy(ei_hbm.at[0, pl.ds(start + q, 1)],
                                src_v.at[pl.ds(q, 1)])
                pltpu.sync_copy(ei_hbm.at[1, pl.ds(start + q, 1)],
                                dst_v.at[pl.ds(q, 1)])

        g_core = g_hbm.at[cid]

        nfull = kb_dyn // NBUF
        for p in range(NBUF):
            pltpu.async_copy(g_core.at[src_v.at[p]], bufs[p], gsems[p])

        def body(gi, carry):
            base = NBUF * gi
            for p in range(NBUF):
                pltpu.make_async_copy(
                    g_core.at[src_v.at[base + p]], bufs[p], gsems[p]).wait()
                pltpu.async_copy(
                    bufs[p], acc_sh.at[dst_v.at[base + p]], ssems[p],
                    add=True)

            @pl.when(gi < nfull - 1)
            def _():
                for p in range(NBUF):
                    pltpu.make_async_copy(
                        bufs[p], acc_sh.at[dst_v.at[base + p]],
                        ssems[p]).wait()
                    pltpu.async_copy(
                        g_core.at[src_v.at[base + NBUF + p]], bufs[p],
                        gsems[p])
            return carry
        lax.fori_loop(0, nfull, body, 0)

        for p in range(NBUF):
            pltpu.make_async_copy(
                bufs[p], acc_sh.at[dst_v.at[0]], ssems[p]).wait()

        def tail(bi, carry):
            pltpu.async_copy(g_core.at[src_v.at[bi]], bufs[0], gsems[0])
            pltpu.make_async_copy(
                g_core.at[src_v.at[bi]], bufs[0], gsems[0]).wait()
            pltpu.sync_copy(bufs[0], acc_sh.at[dst_v.at[bi]], add=True)
            return carry
        lax.fori_loop(nfull * NBUF, kb_dyn, tail, 0)

        plsc.subcore_barrier()
        pltpu.sync_copy(
            acc_sh.at[pl.ds(sid * stripe, stripe)],
            out_hbm.at[pl.ds(sid * stripe, stripe), pl.ds(cid * CH, CH)])

    return k(g2, ei3)


def _final_body(q_ref, g_ref, dis_ref, b_ref, o_ref):
    s = dis_ref[:, :1]
    full = jnp.concatenate(
        [q_ref[:, :CH] + g_ref[0], q_ref[:, CH:] + g_ref[1]], axis=1)
    o_ref[...] = jnp.maximum(full * s + b_ref[:1], 0.0)


def _tc_final(q, g2, dis, bias, n):
    rb_f = max(r for r in (2048, 2000, 1024, 512, 400, 256, 128, 16, 8)
               if n % r == 0)
    return pl.pallas_call(
        _final_body,
        grid=(n // rb_f,),
        in_specs=[pl.BlockSpec((rb_f, C_LANES), lambda i: (i, 0)),
                  pl.BlockSpec((NC, rb_f, CH), lambda i: (0, i, 0)),
                  pl.BlockSpec((rb_f, 8), lambda i: (i, 0)),
                  pl.BlockSpec((8, C_LANES), lambda i: (0, 0))],
        out_specs=pl.BlockSpec((rb_f, C_LANES), lambda i: (i, 0)),
        out_shape=jax.ShapeDtypeStruct((n, C_LANES), jnp.float32),
    )(q, g2, dis, bias)


def kernel(x, edge_index, W, b):
    n, c = x.shape
    e = edge_index.shape[1]
    assert c == C_LANES

    align = 2048
    np_rows = -(-(n + 1) // align) * align
    assert e % (NC * B) == 0, "edge count must be a multiple of 256"
    ei3 = edge_index.astype(jnp.int32).reshape(2, e // B, B)

    deg = _sc_degree(ei3, np_rows)
    g2, dis = _tc_gscale(x, W, deg, np_rows)
    q = _sc_messages(g2, ei3, np_rows)
    bias = jnp.broadcast_to(b.reshape(1, C_LANES), (8, C_LANES))
    return _tc_final(q, g2, dis, bias, n)
